# baseline (speedup 1.0000x reference)
import functools

import jax
import jax.numpy as jnp
from jax import lax
from jax.experimental import pallas as pl
from jax.experimental.pallas import tpu as pltpu
from jax.experimental.pallas import tpu_sc as plsc

N = 10000
E = 320000
D = 128
EPS = 1e-5

NC = 2
NS = 16
NW = NC * NS
EPW = E // NW
C = 80
NCHUNK = EPW // C
NB = 4
RPT = 624
TAIL = N - NS * RPT
ZR = 16


_sc_mesh = plsc.VectorSubcoreMesh(core_axis_name="c", subcore_axis_name="s")


@functools.partial(
    pl.kernel,
    mesh=_sc_mesh,
    out_type=jax.ShapeDtypeStruct((2 * N, D), jnp.float32),
    scratch_types=[
        *[pltpu.VMEM((C,), jnp.int32) for _ in range(NB)],
        *[pltpu.VMEM((C,), jnp.int32) for _ in range(NB)],
        *[pltpu.VMEM((C * 16,), jnp.float32) for _ in range(NB)],
        *[pltpu.VMEM((C, D), jnp.float32) for _ in range(NB)],
        pltpu.VMEM((ZR, D), jnp.float32),
        pltpu.VMEM_SHARED((N, D), jnp.float32),
        pltpu.SemaphoreType.DMA((NB,)),
        pltpu.SemaphoreType.DMA((NB,)),
        pltpu.SemaphoreType.DMA((NB,)),
    ],
)
def _sc_agg(x_hbm, src_hbm, dst_hbm, w_hbm, out_hbm,
            src_0, src_1, src_2, src_3,
            dst_0, dst_1, dst_2, dst_3,
            w_0, w_1, w_2, w_3,
            rows_0, rows_1, rows_2, rows_3,
            zero_v, acc_sh,
            gsem, wsem, ssem):
    src_slots = (src_0, src_1, src_2, src_3)
    dst_slots = (dst_0, dst_1, dst_2, dst_3)
    w_slots = (w_0, w_1, w_2, w_3)
    rows_slots = (rows_0, rows_1, rows_2, rows_3)
    cid = lax.axis_index("c")
    sid = lax.axis_index("s")
    wid = sid * NC + cid

    def _zrow(r, carry):
        for k in range(D // 16):
            zero_v[r, pl.ds(k * 16, 16)] = jnp.zeros((16,), jnp.float32)
        return carry

    lax.fori_loop(0, ZR, _zrow, 0)

    def _zacc(t, carry):
        pltpu.sync_copy(zero_v, acc_sh.at[pl.ds(sid * RPT + t * ZR, ZR)])
        return carry

    lax.fori_loop(0, RPT // ZR, _zacc, 0)

    @pl.when(sid == 0)
    def _():
        pltpu.sync_copy(zero_v.at[pl.ds(0, TAIL)],
                        acc_sh.at[pl.ds(NS * RPT, TAIL)])

    plsc.subcore_barrier()

    def _start_fetch(i, b):
        base = wid * EPW + i * C
        pltpu.make_async_copy(src_hbm.at[pl.ds(base, C)], src_slots[b],
                              wsem.at[b]).start()
        pltpu.make_async_copy(dst_hbm.at[pl.ds(base, C)], dst_slots[b],
                              wsem.at[b]).start()
        pltpu.make_async_copy(w_hbm.at[pl.ds(base * 16, C * 16)], w_slots[b],
                              wsem.at[b]).start()

    def _drain_fetch(b):
        pltpu.make_async_copy(src_hbm.at[pl.ds(0, C)], src_slots[b],
                              wsem.at[b]).wait()
        pltpu.make_async_copy(dst_hbm.at[pl.ds(0, C)], dst_slots[b],
                              wsem.at[b]).wait()
        pltpu.make_async_copy(w_hbm.at[pl.ds(0, C * 16)], w_slots[b],
                              wsem.at[b]).wait()

    def _start_gather(b):
        pltpu.make_async_copy(x_hbm.at[src_slots[b]], rows_slots[b],
                              gsem.at[b]).start()

    def _drain_gather(b):
        pltpu.make_async_copy(x_hbm.at[pl.ds(0, C)], rows_slots[b],
                              gsem.at[b]).wait()

    def _drain_scatter(b):
        pltpu.make_async_copy(x_hbm.at[pl.ds(0, C)], rows_slots[b],
                              ssem.at[b]).wait()

    def _mul1(j, b):
        wrow = w_slots[b][pl.ds(j * 16, 16)]
        rv = rows_slots[b]
        for k in range(D // 16):
            sl = (j, pl.ds(k * 16, 16))
            rv[sl] = rv[sl] * wrow

    def _process(i, b):
        _drain_gather(b)

        def _mul_body(jj, carry2):
            for e in range(8):
                _mul1(jj * 8 + e, b)
            return carry2

        lax.fori_loop(0, C // 8, _mul_body, 0)
        pltpu.make_async_copy(rows_slots[b], acc_sh.at[dst_slots[b]],
                              ssem.at[b]).start(add=True)

    _start_fetch(0, 0)
    _start_fetch(1, 1)
    _drain_fetch(0)
    _start_gather(0)

    def _group(gi, carry):
        for b in range(NB):
            i = gi * NB + b
            b1 = (b + 1) % NB
            b2 = (b + 2) % NB

            @pl.when(i >= 2)
            def _():
                _drain_scatter(b2)

            @pl.when(i + 2 < NCHUNK)
            def _():
                _start_fetch(i + 2, b2)

            _drain_fetch(b1)
            _start_gather(b1)
            _process(i, b)
        return carry

    lax.fori_loop(0, (NCHUNK - 1) // NB, _group, 0)
    _process(NCHUNK - 1, (NCHUNK - 1) % NB)
    for b in ((NCHUNK - 3) % NB, (NCHUNK - 2) % NB, (NCHUNK - 1) % NB):
        _drain_scatter(b)
    plsc.subcore_barrier()

    pltpu.sync_copy(acc_sh.at[pl.ds(sid * RPT, RPT)],
                    out_hbm.at[pl.ds(cid * N + sid * RPT, RPT)])

    @pl.when(sid == 0)
    def _():
        pltpu.sync_copy(acc_sh.at[pl.ds(NS * RPT, TAIL)],
                        out_hbm.at[pl.ds(cid * N + NS * RPT, TAIL)])


def _mm_body(x_ref, wt_ref, w2_ref, rbig_ref, o_ref, wexp_ref):
    o_ref[...] = jnp.dot(x_ref[...], wt_ref[...],
                         preferred_element_type=jnp.float32)
    wexp_ref[...] = jnp.dot(w2_ref[...], rbig_ref[...],
                            preferred_element_type=jnp.float32)


def _pre_mm(x, wt, w2, rbig):
    return pl.pallas_call(
        _mm_body,
        out_shape=[jax.ShapeDtypeStruct((N, D), jnp.float32),
                   jax.ShapeDtypeStruct((E // 128, 2048), jnp.float32)],
    )(x, wt, w2, rbig)


def _mid_body(p_ref, b_ref, g_ref, be_ref, wt_ref, o_ref):
    y = p_ref[0] + p_ref[1] + b_ref[...]
    mean = jnp.mean(y, axis=0, keepdims=True)
    var = jnp.mean((y - mean) ** 2, axis=0, keepdims=True)
    h = (y - mean) * lax.rsqrt(var + EPS) * g_ref[...] + be_ref[...]
    h = jnp.maximum(h, 0.0)
    o_ref[...] = jnp.dot(h, wt_ref[...], preferred_element_type=jnp.float32)


def _mid(p, b, g, be, wt):
    return pl.pallas_call(
        _mid_body,
        out_shape=jax.ShapeDtypeStruct((N, D), jnp.float32),
    )(p, b.reshape(1, D), g.reshape(1, D), be.reshape(1, D), wt)


def _post_body(p_ref, b_ref, g_ref, be_ref, o_ref):
    y = p_ref[0] + p_ref[1] + b_ref[...]
    mean = jnp.mean(y, axis=0, keepdims=True)
    var = jnp.mean((y - mean) ** 2, axis=0, keepdims=True)
    h = (y - mean) * lax.rsqrt(var + EPS) * g_ref[...] + be_ref[...]
    o_ref[...] = jnp.maximum(h, 0.0)


def _post(p, b, g, be):
    return pl.pallas_call(
        _post_body,
        out_shape=jax.ShapeDtypeStruct((N, D), jnp.float32),
    )(p, b.reshape(1, D), g.reshape(1, D), be.reshape(1, D))


def kernel(node_features, edge_index, edge_weights,
           W0, b0, g0, be0, W1, b1, g1, be1):
    x0 = node_features[:, 0, :]
    src = edge_index[0].astype(jnp.int32)
    dst = edge_index[1].astype(jnp.int32)
    rbig = (lax.broadcasted_iota(jnp.int32, (D, 2048), 0)
            == lax.broadcasted_iota(jnp.int32, (D, 2048), 1) // 16
            ).astype(jnp.float32)
    w2 = edge_weights[0].reshape(E // D, D)

    xw0, wexp = _pre_mm(x0, W0.T, w2, rbig)
    w = wexp.reshape(E * 16)
    p0 = _sc_agg(xw0, src, dst, w).reshape(2, N, D)
    xw1 = _mid(p0, b0, g0, be0, W1.T)
    p1 = _sc_agg(xw1, src, dst, w).reshape(2, N, D)
    out = _post(p1, b1, g1, be1)
    return out[:, None, :]


# speedup vs baseline: 1.1092x; 1.1092x over previous
import functools

import jax
import jax.numpy as jnp
from jax import lax
from jax.experimental import pallas as pl
from jax.experimental.pallas import tpu as pltpu
from jax.experimental.pallas import tpu_sc as plsc

N = 10000
E = 320000
D = 128
EPS = 1e-5

NC = 2
NS = 16
NW = NC * NS
EPW = E // NW
C = 40
NCHUNK = EPW // C
NB = 5
PF = 3
RPT = 624
TAIL = N - NS * RPT
ZR = 16


_sc_mesh = plsc.VectorSubcoreMesh(core_axis_name="c", subcore_axis_name="s")


@functools.partial(
    pl.kernel,
    mesh=_sc_mesh,
    out_type=jax.ShapeDtypeStruct((2 * N, D), jnp.float32),
    scratch_types=[
        pltpu.VMEM((EPW,), jnp.int32),
        *[pltpu.VMEM((C,), jnp.int32) for _ in range(NB)],
        *[pltpu.VMEM((C * 16,), jnp.float32) for _ in range(NB)],
        *[pltpu.VMEM((C, D), jnp.float32) for _ in range(NB)],
        pltpu.VMEM((ZR, D), jnp.float32),
        pltpu.VMEM_SHARED((N, D), jnp.float32),
        pltpu.SemaphoreType.DMA((NB,)),
        pltpu.SemaphoreType.DMA((NB,)),
        pltpu.SemaphoreType.DMA((NB,)),
    ],
)
def _sc_agg(x_hbm, src_hbm, dst_hbm, w_hbm, out_hbm,
            src_all,
            dst_0, dst_1, dst_2, dst_3, dst_4,
            w_0, w_1, w_2, w_3, w_4,
            rows_0, rows_1, rows_2, rows_3, rows_4,
            zero_v, acc_sh,
            gsem, wsem, ssem):
    dst_slots = (dst_0, dst_1, dst_2, dst_3, dst_4)
    w_slots = (w_0, w_1, w_2, w_3, w_4)
    rows_slots = (rows_0, rows_1, rows_2, rows_3, rows_4)
    cid = lax.axis_index("c")
    sid = lax.axis_index("s")
    wid = sid * NC + cid

    pltpu.sync_copy(src_hbm.at[pl.ds(wid * EPW, EPW)], src_all)

    def _zrow(r, carry):
        for k in range(D // 16):
            zero_v[r, pl.ds(k * 16, 16)] = jnp.zeros((16,), jnp.float32)
        return carry

    lax.fori_loop(0, ZR, _zrow, 0)

    def _zacc(t, carry):
        pltpu.sync_copy(zero_v, acc_sh.at[pl.ds(sid * RPT + t * ZR, ZR)])
        return carry

    lax.fori_loop(0, RPT // ZR, _zacc, 0)

    @pl.when(sid == 0)
    def _():
        pltpu.sync_copy(zero_v.at[pl.ds(0, TAIL)],
                        acc_sh.at[pl.ds(NS * RPT, TAIL)])

    def _start_fetch(i, b):
        base = wid * EPW + i * C
        pltpu.make_async_copy(dst_hbm.at[pl.ds(base, C)], dst_slots[b],
                              wsem.at[b]).start()
        pltpu.make_async_copy(w_hbm.at[pl.ds(base * 16, C * 16)], w_slots[b],
                              wsem.at[b]).start()
        pltpu.make_async_copy(x_hbm.at[src_all.at[pl.ds(i * C, C)]],
                              rows_slots[b], gsem.at[b]).start()

    def _drain_gather(b):
        pltpu.make_async_copy(dst_hbm.at[pl.ds(0, C)], dst_slots[b],
                              wsem.at[b]).wait()
        pltpu.make_async_copy(w_hbm.at[pl.ds(0, C * 16)], w_slots[b],
                              wsem.at[b]).wait()
        pltpu.make_async_copy(x_hbm.at[pl.ds(0, C)], rows_slots[b],
                              gsem.at[b]).wait()

    def _drain_scatter(b):
        pltpu.make_async_copy(x_hbm.at[pl.ds(0, C)], rows_slots[b],
                              ssem.at[b]).wait()

    for j in range(PF):
        _start_fetch(j, j)
    plsc.subcore_barrier()

    def _mul1(j, b):
        wrow = w_slots[b][pl.ds(j * 16, 16)]
        rv = rows_slots[b]
        for k in range(D // 16):
            sl = (j, pl.ds(k * 16, 16))
            rv[sl] = rv[sl] * wrow

    def _group(gi, carry):
        for b in range(NB):
            i = gi * NB + b
            pb = (b + PF) % NB

            @pl.when(i + PF < NCHUNK)
            def _():
                @pl.when(i >= NB - PF)
                def _():
                    _drain_scatter(pb)
                _start_fetch(i + PF, pb)

            _drain_gather(b)

            def _mul_body(jj, carry2):
                for e in range(8):
                    _mul1(jj * 8 + e, b)
                return carry2

            lax.fori_loop(0, C // 8, _mul_body, 0)
            pltpu.make_async_copy(rows_slots[b], acc_sh.at[dst_slots[b]],
                                  ssem.at[b]).start(add=True)
        return carry

    lax.fori_loop(0, NCHUNK // NB, _group, 0)
    for b in range(NB):
        _drain_scatter(b)
    plsc.subcore_barrier()

    pltpu.sync_copy(acc_sh.at[pl.ds(sid * RPT, RPT)],
                    out_hbm.at[pl.ds(cid * N + sid * RPT, RPT)])

    @pl.when(sid == 0)
    def _():
        pltpu.sync_copy(acc_sh.at[pl.ds(NS * RPT, TAIL)],
                        out_hbm.at[pl.ds(cid * N + NS * RPT, TAIL)])


def _mm_body(x_ref, wt_ref, w2_ref, rbig_ref, o_ref, wexp_ref):
    o_ref[...] = jnp.dot(x_ref[...], wt_ref[...],
                         preferred_element_type=jnp.float32)
    wexp_ref[...] = jnp.dot(w2_ref[...], rbig_ref[...],
                            preferred_element_type=jnp.float32)


def _pre_mm(x, wt, w2, rbig):
    return pl.pallas_call(
        _mm_body,
        out_shape=[jax.ShapeDtypeStruct((N, D), jnp.float32),
                   jax.ShapeDtypeStruct((E // 128, 2048), jnp.float32)],
    )(x, wt, w2, rbig)


def _mid_body(p_ref, b_ref, g_ref, be_ref, wt_ref, o_ref):
    y = p_ref[0] + p_ref[1] + b_ref[...]
    mean = jnp.mean(y, axis=0, keepdims=True)
    var = jnp.mean((y - mean) ** 2, axis=0, keepdims=True)
    h = (y - mean) * lax.rsqrt(var + EPS) * g_ref[...] + be_ref[...]
    h = jnp.maximum(h, 0.0)
    o_ref[...] = jnp.dot(h, wt_ref[...], preferred_element_type=jnp.float32)


def _mid(p, b, g, be, wt):
    return pl.pallas_call(
        _mid_body,
        out_shape=jax.ShapeDtypeStruct((N, D), jnp.float32),
    )(p, b.reshape(1, D), g.reshape(1, D), be.reshape(1, D), wt)


def _post_body(p_ref, b_ref, g_ref, be_ref, o_ref):
    y = p_ref[0] + p_ref[1] + b_ref[...]
    mean = jnp.mean(y, axis=0, keepdims=True)
    var = jnp.mean((y - mean) ** 2, axis=0, keepdims=True)
    h = (y - mean) * lax.rsqrt(var + EPS) * g_ref[...] + be_ref[...]
    o_ref[...] = jnp.maximum(h, 0.0)


def _post(p, b, g, be):
    return pl.pallas_call(
        _post_body,
        out_shape=jax.ShapeDtypeStruct((N, D), jnp.float32),
    )(p, b.reshape(1, D), g.reshape(1, D), be.reshape(1, D))


def kernel(node_features, edge_index, edge_weights,
           W0, b0, g0, be0, W1, b1, g1, be1):
    x0 = node_features[:, 0, :]
    src = edge_index[0].astype(jnp.int32)
    dst = edge_index[1].astype(jnp.int32)
    rbig = (lax.broadcasted_iota(jnp.int32, (D, 2048), 0)
            == lax.broadcasted_iota(jnp.int32, (D, 2048), 1) // 16
            ).astype(jnp.float32)
    w2 = edge_weights[0].reshape(E // D, D)

    xw0, wexp = _pre_mm(x0, W0.T, w2, rbig)
    w = wexp.reshape(E * 16)
    p0 = _sc_agg(xw0, src, dst, w).reshape(2, N, D)
    xw1 = _mid(p0, b0, g0, be0, W1.T)
    p1 = _sc_agg(xw1, src, dst, w).reshape(2, N, D)
    out = _post(p1, b1, g1, be1)
    return out[:, None, :]


# speedup vs baseline: 1.1802x; 1.0641x over previous
import functools

import jax
import jax.numpy as jnp
from jax import lax
from jax.experimental import pallas as pl
from jax.experimental.pallas import tpu as pltpu
from jax.experimental.pallas import tpu_sc as plsc

N = 10000
E = 320000
D = 128
EPS = 1e-5

NC = 2
NS = 16
NW = NC * NS
EPW = E // NW
C = 40
NCHUNK = EPW // C
NB = 5
PF = 3
RPT = 624
TAIL = N - NS * RPT
ZR = 16


_sc_mesh = plsc.VectorSubcoreMesh(core_axis_name="c", subcore_axis_name="s")


@functools.partial(
    pl.kernel,
    mesh=_sc_mesh,
    out_type=jax.ShapeDtypeStruct((2 * N, D), jnp.float32),
    scratch_types=[
        pltpu.VMEM((EPW,), jnp.int32),
        *[pltpu.VMEM((C,), jnp.int32) for _ in range(NB)],
        *[pltpu.VMEM((C * 16,), jnp.float32) for _ in range(NB)],
        *[pltpu.VMEM((C, D), jnp.float32) for _ in range(NB)],
        pltpu.VMEM((ZR, D), jnp.float32),
        pltpu.VMEM_SHARED((N, D), jnp.float32),
        pltpu.SemaphoreType.DMA((NB,)),
        pltpu.SemaphoreType.DMA((NB,)),
        pltpu.SemaphoreType.DMA((NB,)),
    ],
)
def _sc_agg(x_hbm, src_hbm, dst_hbm, w_hbm, out_hbm,
            src_all,
            dst_0, dst_1, dst_2, dst_3, dst_4,
            w_0, w_1, w_2, w_3, w_4,
            rows_0, rows_1, rows_2, rows_3, rows_4,
            zero_v, acc_sh,
            gsem, wsem, ssem):
    dst_slots = (dst_0, dst_1, dst_2, dst_3, dst_4)
    w_slots = (w_0, w_1, w_2, w_3, w_4)
    rows_slots = (rows_0, rows_1, rows_2, rows_3, rows_4)
    cid = lax.axis_index("c")
    sid = lax.axis_index("s")
    wid = sid * NC + cid

    pltpu.sync_copy(src_hbm.at[pl.ds(wid * EPW, EPW)], src_all)

    def _zrow(r, carry):
        for k in range(D // 16):
            zero_v[r, pl.ds(k * 16, 16)] = jnp.zeros((16,), jnp.float32)
        return carry

    lax.fori_loop(0, ZR, _zrow, 0)

    def _zacc(t, carry):
        pltpu.sync_copy(zero_v, acc_sh.at[pl.ds(sid * RPT + t * ZR, ZR)])
        return carry

    lax.fori_loop(0, RPT // ZR, _zacc, 0)

    @pl.when(sid == 0)
    def _():
        pltpu.sync_copy(zero_v.at[pl.ds(0, TAIL)],
                        acc_sh.at[pl.ds(NS * RPT, TAIL)])

    def _start_fetch(i, b):
        base = wid * EPW + i * C
        pltpu.make_async_copy(dst_hbm.at[pl.ds(base, C)], dst_slots[b],
                              wsem.at[b]).start()
        pltpu.make_async_copy(w_hbm.at[pl.ds(base * 16, C * 16)], w_slots[b],
                              wsem.at[b]).start()
        pltpu.make_async_copy(x_hbm.at[src_all.at[pl.ds(i * C, C)]],
                              rows_slots[b], gsem.at[b]).start()

    def _drain_gather(b):
        pltpu.make_async_copy(dst_hbm.at[pl.ds(0, C)], dst_slots[b],
                              wsem.at[b]).wait()
        pltpu.make_async_copy(w_hbm.at[pl.ds(0, C * 16)], w_slots[b],
                              wsem.at[b]).wait()
        pltpu.make_async_copy(x_hbm.at[pl.ds(0, C)], rows_slots[b],
                              gsem.at[b]).wait()

    def _drain_scatter(b):
        pltpu.make_async_copy(x_hbm.at[pl.ds(0, C)], rows_slots[b],
                              ssem.at[b]).wait()

    for j in range(PF):
        _start_fetch(j, j)
    plsc.subcore_barrier()

    def _mul1(j, b):
        wrow = w_slots[b][pl.ds(j * 16, 16)]
        rv = rows_slots[b]
        for k in range(D // 16):
            sl = (j, pl.ds(k * 16, 16))
            rv[sl] = rv[sl] * wrow

    def _group(gi, carry):
        for b in range(NB):
            i = gi * NB + b
            pb = (b + PF) % NB

            @pl.when(i + PF < NCHUNK)
            def _():
                @pl.when(i >= NB - PF)
                def _():
                    _drain_scatter(pb)
                _start_fetch(i + PF, pb)

            _drain_gather(b)

            def _mul_body(jj, carry2):
                for e in range(8):
                    _mul1(jj * 8 + e, b)
                return carry2

            lax.fori_loop(0, C // 8, _mul_body, 0)
            pltpu.make_async_copy(rows_slots[b], acc_sh.at[dst_slots[b]],
                                  ssem.at[b]).start(add=True)
        return carry

    lax.fori_loop(0, NCHUNK // NB, _group, 0)
    for b in range(NB):
        _drain_scatter(b)
    plsc.subcore_barrier()

    pltpu.sync_copy(acc_sh.at[pl.ds(sid * RPT, RPT)],
                    out_hbm.at[pl.ds(cid * N + sid * RPT, RPT)])

    @pl.when(sid == 0)
    def _():
        pltpu.sync_copy(acc_sh.at[pl.ds(NS * RPT, TAIL)],
                        out_hbm.at[pl.ds(cid * N + NS * RPT, TAIL)])


def _mm_body(x_ref, wt_ref, w2_ref, rbig_ref, o_ref, wexp_ref):
    o_ref[...] = jnp.dot(x_ref[...], wt_ref[...],
                         preferred_element_type=jnp.float32)
    wexp_ref[...] = jnp.dot(w2_ref[...], rbig_ref[...],
                            preferred_element_type=jnp.float32).reshape(E * 16)


def _pre_mm(x, wt, w2, rbig):
    return pl.pallas_call(
        _mm_body,
        out_shape=[jax.ShapeDtypeStruct((N, D), jnp.float32),
                   jax.ShapeDtypeStruct((E * 16,), jnp.float32)],
    )(x, wt, w2, rbig)


def _mid_body(p_ref, b_ref, g_ref, be_ref, wt_ref, o_ref):
    y = p_ref[0] + p_ref[1] + b_ref[...]
    mean = jnp.mean(y, axis=0, keepdims=True)
    var = jnp.mean((y - mean) ** 2, axis=0, keepdims=True)
    h = (y - mean) * lax.rsqrt(var + EPS) * g_ref[...] + be_ref[...]
    h = jnp.maximum(h, 0.0)
    o_ref[...] = jnp.dot(h, wt_ref[...], preferred_element_type=jnp.float32)


def _mid(p, b, g, be, wt):
    return pl.pallas_call(
        _mid_body,
        out_shape=jax.ShapeDtypeStruct((N, D), jnp.float32),
    )(p, b.reshape(1, D), g.reshape(1, D), be.reshape(1, D), wt)


def _post_body(p_ref, b_ref, g_ref, be_ref, o_ref):
    y = p_ref[0] + p_ref[1] + b_ref[...]
    mean = jnp.mean(y, axis=0, keepdims=True)
    var = jnp.mean((y - mean) ** 2, axis=0, keepdims=True)
    h = (y - mean) * lax.rsqrt(var + EPS) * g_ref[...] + be_ref[...]
    o_ref[...] = jnp.maximum(h, 0.0)


def _post(p, b, g, be):
    return pl.pallas_call(
        _post_body,
        out_shape=jax.ShapeDtypeStruct((N, D), jnp.float32),
    )(p, b.reshape(1, D), g.reshape(1, D), be.reshape(1, D))


def kernel(node_features, edge_index, edge_weights,
           W0, b0, g0, be0, W1, b1, g1, be1):
    x0 = node_features[:, 0, :]
    src = edge_index[0].astype(jnp.int32)
    dst = edge_index[1].astype(jnp.int32)
    rbig = (lax.broadcasted_iota(jnp.int32, (D, 2048), 0)
            == lax.broadcasted_iota(jnp.int32, (D, 2048), 1) // 16
            ).astype(jnp.float32)
    w2 = edge_weights[0].reshape(E // D, D)

    xw0, w = _pre_mm(x0, W0.T, w2, rbig)
    p0 = _sc_agg(xw0, src, dst, w).reshape(2, N, D)
    xw1 = _mid(p0, b0, g0, be0, W1.T)
    p1 = _sc_agg(xw1, src, dst, w).reshape(2, N, D)
    out = _post(p1, b1, g1, be1)
    return out[:, None, :]


# speedup vs baseline: 1.1802x; 1.0000x over previous
import functools

import jax
import jax.numpy as jnp
from jax import lax
from jax.experimental import pallas as pl
from jax.experimental.pallas import tpu as pltpu
from jax.experimental.pallas import tpu_sc as plsc

N = 10000
E = 320000
D = 128
EPS = 1e-5

NC = 2
NS = 16
NW = NC * NS
EPW = E // NW
C = 40
NCHUNK = EPW // C
NB = 5
PF = 3
RPT = 624
TAIL = N - NS * RPT
ZR = 16


_sc_mesh = plsc.VectorSubcoreMesh(core_axis_name="c", subcore_axis_name="s")


@functools.partial(
    pl.kernel,
    mesh=_sc_mesh,
    out_type=jax.ShapeDtypeStruct((2 * N, D), jnp.float32),
    scratch_types=[
        pltpu.VMEM((EPW,), jnp.int32),
        *[pltpu.VMEM((C,), jnp.int32) for _ in range(NB)],
        *[pltpu.VMEM((C * 16,), jnp.float32) for _ in range(NB)],
        *[pltpu.VMEM((C, D), jnp.float32) for _ in range(NB)],
        pltpu.VMEM((ZR, D), jnp.float32),
        pltpu.VMEM_SHARED((N, D), jnp.float32),
        pltpu.SemaphoreType.DMA((NB,)),
        pltpu.SemaphoreType.DMA((NB,)),
        pltpu.SemaphoreType.DMA((NB,)),
    ],
)
def _sc_agg(x_hbm, src_hbm, dst_hbm, w_hbm, out_hbm,
            src_all,
            dst_0, dst_1, dst_2, dst_3, dst_4,
            w_0, w_1, w_2, w_3, w_4,
            rows_0, rows_1, rows_2, rows_3, rows_4,
            zero_v, acc_sh,
            gsem, wsem, ssem):
    dst_slots = (dst_0, dst_1, dst_2, dst_3, dst_4)
    w_slots = (w_0, w_1, w_2, w_3, w_4)
    rows_slots = (rows_0, rows_1, rows_2, rows_3, rows_4)
    cid = lax.axis_index("c")
    sid = lax.axis_index("s")
    wid = sid * NC + cid

    pltpu.sync_copy(src_hbm.at[pl.ds(wid * EPW, EPW)], src_all)

    def _zrow(r, carry):
        for k in range(D // 16):
            zero_v[r, pl.ds(k * 16, 16)] = jnp.zeros((16,), jnp.float32)
        return carry

    lax.fori_loop(0, ZR, _zrow, 0)

    def _zacc(t, carry):
        pltpu.sync_copy(zero_v, acc_sh.at[pl.ds(sid * RPT + t * ZR, ZR)])
        return carry

    lax.fori_loop(0, RPT // ZR, _zacc, 0)

    @pl.when(sid == 0)
    def _():
        pltpu.sync_copy(zero_v.at[pl.ds(0, TAIL)],
                        acc_sh.at[pl.ds(NS * RPT, TAIL)])

    def _start_fetch(i, b):
        base = wid * EPW + i * C
        pltpu.make_async_copy(dst_hbm.at[pl.ds(base, C)], dst_slots[b],
                              wsem.at[b]).start()
        pltpu.make_async_copy(w_hbm.at[pl.ds(base * 16, C * 16)], w_slots[b],
                              wsem.at[b]).start()
        pltpu.make_async_copy(x_hbm.at[src_all.at[pl.ds(i * C, C)]],
                              rows_slots[b], gsem.at[b]).start()

    def _drain_gather(b):
        pltpu.make_async_copy(dst_hbm.at[pl.ds(0, C)], dst_slots[b],
                              wsem.at[b]).wait()
        pltpu.make_async_copy(w_hbm.at[pl.ds(0, C * 16)], w_slots[b],
                              wsem.at[b]).wait()
        pltpu.make_async_copy(x_hbm.at[pl.ds(0, C)], rows_slots[b],
                              gsem.at[b]).wait()

    def _drain_scatter(b):
        pltpu.make_async_copy(x_hbm.at[pl.ds(0, C)], rows_slots[b],
                              ssem.at[b]).wait()

    for j in range(PF):
        _start_fetch(j, j)
    plsc.subcore_barrier()

    def _mul1(j, b):
        wrow = w_slots[b][pl.ds(j * 16, 16)]
        rv = rows_slots[b]
        for k in range(D // 16):
            sl = (j, pl.ds(k * 16, 16))
            rv[sl] = rv[sl] * wrow

    def _group(gi, carry):
        for b in range(NB):
            i = gi * NB + b
            pb = (b + PF) % NB

            @pl.when(i + PF < NCHUNK)
            def _():
                @pl.when(i >= NB - PF)
                def _():
                    _drain_scatter(pb)
                _start_fetch(i + PF, pb)

            _drain_gather(b)

            def _mul_body(jj, carry2):
                for e in range(8):
                    _mul1(jj * 8 + e, b)
                return carry2

            lax.fori_loop(0, C // 8, _mul_body, 0)
            pltpu.make_async_copy(rows_slots[b], acc_sh.at[dst_slots[b]],
                                  ssem.at[b]).start(add=True)
        return carry

    lax.fori_loop(0, NCHUNK // NB, _group, 0)
    for b in range(NB):
        _drain_scatter(b)
    plsc.subcore_barrier()

    pltpu.sync_copy(acc_sh.at[pl.ds(sid * RPT, RPT)],
                    out_hbm.at[pl.ds(cid * N + sid * RPT, RPT)])

    @pl.when(sid == 0)
    def _():
        pltpu.sync_copy(acc_sh.at[pl.ds(NS * RPT, TAIL)],
                        out_hbm.at[pl.ds(cid * N + NS * RPT, TAIL)])


def _mm_body(x_ref, wt_ref, w2_ref, rbig_ref, o_ref, wexp_ref):
    o_ref[...] = jnp.dot(x_ref[...], wt_ref[...],
                         preferred_element_type=jnp.float32)
    wexp_ref[...] = jnp.dot(w2_ref[...], rbig_ref[...],
                            preferred_element_type=jnp.float32).reshape(E * 16)


def _pre_mm(x, wt, w2, rbig):
    return pl.pallas_call(
        _mm_body,
        out_shape=[jax.ShapeDtypeStruct((N, D), jnp.float32),
                   jax.ShapeDtypeStruct((E * 16,), jnp.float32)],
    )(x, wt, w2, rbig)


def _mid_body(p_ref, b_ref, g_ref, be_ref, wt_ref, o_ref):
    y = p_ref[0] + p_ref[1] + b_ref[...]
    mean = jnp.mean(y, axis=0, keepdims=True)
    var = jnp.mean((y - mean) ** 2, axis=0, keepdims=True)
    h = (y - mean) * lax.rsqrt(var + EPS) * g_ref[...] + be_ref[...]
    h = jnp.maximum(h, 0.0)
    o_ref[...] = jnp.dot(h, wt_ref[...], preferred_element_type=jnp.float32)


def _mid(p, b, g, be, wt):
    return pl.pallas_call(
        _mid_body,
        out_shape=jax.ShapeDtypeStruct((N, D), jnp.float32),
    )(p, b.reshape(1, D), g.reshape(1, D), be.reshape(1, D), wt)


def _post_body(p_ref, b_ref, g_ref, be_ref, o_ref):
    y = p_ref[0] + p_ref[1] + b_ref[...]
    mean = jnp.mean(y, axis=0, keepdims=True)
    var = jnp.mean((y - mean) ** 2, axis=0, keepdims=True)
    h = (y - mean) * lax.rsqrt(var + EPS) * g_ref[...] + be_ref[...]
    o_ref[...] = jnp.maximum(h, 0.0)


def _post(p, b, g, be):
    return pl.pallas_call(
        _post_body,
        out_shape=jax.ShapeDtypeStruct((N, D), jnp.float32),
    )(p, b.reshape(1, D), g.reshape(1, D), be.reshape(1, D))


def kernel(node_features, edge_index, edge_weights,
           W0, b0, g0, be0, W1, b1, g1, be1):
    x0 = node_features[:, 0, :]
    src = edge_index[0].astype(jnp.int32)
    dst = edge_index[1].astype(jnp.int32)
    rbig = (lax.broadcasted_iota(jnp.int32, (D, 2048), 0)
            == lax.broadcasted_iota(jnp.int32, (D, 2048), 1) // 16
            ).astype(jnp.float32)
    w2 = edge_weights.reshape(E // D, D)

    xw0, w = _pre_mm(x0, W0.T, w2, rbig)
    p0 = _sc_agg(xw0, src, dst, w).reshape(2, N, D)
    xw1 = _mid(p0, b0, g0, be0, W1.T)
    p1 = _sc_agg(xw1, src, dst, w).reshape(2, N, D)
    out = _post(p1, b1, g1, be1)
    return out[:, None, :]
